# baseline (speedup 1.0000x reference)
import functools

import jax
import jax.numpy as jnp
from jax import lax
from jax.experimental import pallas as pl
from jax.experimental.pallas import tpu as pltpu
from jax.experimental.pallas import tpu_sc as plsc

_NC = 2
_NS = 16
_NW = _NC * _NS
_CHUNK = 128


def _vq_tc_body(h_ref, w_ref, idx_ref, loss_ref):
    i = pl.program_id(0)
    hb = h_ref[...]
    w = w_ref[...]
    K = w.shape[0]
    D = hb.shape[1]
    w_sq = jnp.sum(w * w, axis=1, keepdims=True)
    m2 = jax.lax.dot_general(-2.0 * w, hb, (((1,), (1,)), ((), ())),
                             preferred_element_type=jnp.float32)
    score = w_sq + m2
    minval = jnp.min(score, axis=0, keepdims=True)
    iota_f = jax.lax.broadcasted_iota(
        jnp.int32, score.shape, 0).astype(jnp.float32)
    idx_f = jnp.min(jnp.where(score == minval, iota_f, float(K)), axis=0)
    BN = idx_f.shape[0]
    idx_ref[0, :, :] = idx_f.astype(jnp.int32).reshape(BN // 128, 128)

    h_sq = jax.lax.dot_general(
        jnp.ones((1, D), jnp.float32), hb * hb,
        (((1,), (1,)), ((), ())),
        preferred_element_type=jnp.float32)

    @pl.when(i == 0)
    def _():
        loss_ref[0, 0] = 0.0

    loss_ref[0, 0] += jnp.sum(h_sq + minval)


def _tc_stage(h_flat, W, BN, off, rows):
    N, D = h_flat.shape
    K = W.shape[0]
    grid = rows // BN
    off_blocks = off // BN

    idx3, loss_sum = pl.pallas_call(
        _vq_tc_body,
        grid=(grid,),
        in_specs=[
            pl.BlockSpec((BN, D), lambda i: (i + off_blocks, 0)),
            pl.BlockSpec((K, D), lambda i: (0, 0)),
        ],
        out_specs=[
            pl.BlockSpec((1, BN // 128, 128), lambda i: (i, 0, 0)),
            pl.BlockSpec((1, 1), lambda i: (0, 0), memory_space=pltpu.SMEM),
        ],
        out_shape=[
            jax.ShapeDtypeStruct((grid, BN // 128, 128), jnp.int32),
            jax.ShapeDtypeStruct((1, 1), jnp.float32),
        ],
        compiler_params=pltpu.CompilerParams(
            dimension_semantics=("arbitrary",)),
    )(h_flat, W)
    return idx3.reshape(rows), loss_sum


def _make_sc_gather(N, K, D):
    b_per_w = N // _NW
    n_sub = 2
    sub = b_per_w // n_sub
    groups = sub // 16
    mesh = plsc.VectorSubcoreMesh(core_axis_name="c", subcore_axis_name="s")

    @functools.partial(
        pl.kernel,
        mesh=mesh,
        out_type=jax.ShapeDtypeStruct((N * D,), jnp.float32),
        compiler_params=pltpu.CompilerParams(
            use_tc_tiling_on_sc=False, needs_layout_passes=False),
        scratch_types=[
            pltpu.VMEM((b_per_w,), jnp.int32),
            pltpu.VMEM((K * D,), jnp.float32),
            pltpu.VMEM((sub * D,), jnp.float32),
            pltpu.SemaphoreType.DMA,
        ],
    )
    def gather_kernel(idx_hbm, table_hbm, out_hbm, idx_v, tab_v, rows_v,
                      sem):
        wid = lax.axis_index("s") * _NC + lax.axis_index("c")
        base = wid * b_per_w
        pltpu.sync_copy(idx_hbm.at[pl.ds(base, b_per_w)], idx_v)
        pltpu.sync_copy(table_hbm, tab_v)
        lane = jax.lax.broadcasted_iota(jnp.int32, (16,), 0)
        for s in range(n_sub):

            def group_body(g, carry):
                iv = idx_v[pl.ds(s * sub + g * 16, 16)]
                src = iv * D
                dst = (g * 16 + lane) * D
                for c in range(D):
                    vals = plsc.load_gather(tab_v, [src + c])
                    plsc.store_scatter(rows_v, [dst + c], vals)
                return carry

            jax.lax.fori_loop(0, groups, group_body, 0)
            pltpu.sync_copy(
                rows_v, out_hbm.at[pl.ds((base + s * sub) * D, sub * D)])

    return gather_kernel


def kernel(h, W):
    N = h.shape[0] * h.shape[1]
    D = h.shape[2]
    K = W.shape[0]
    h_flat = h.reshape(N, D)

    idx, loss_sum = _tc_stage(h_flat, W, 2048, 0, N)
    q_flat = _make_sc_gather(N, K, D)(idx, W.reshape(K * D))
    q = q_flat.reshape(h.shape)

    mse = loss_sum[0, 0] / jnp.float32(N * D)
    commitment_loss = jnp.float32(0.25) * mse
    codebook_loss = mse
    return q, commitment_loss, codebook_loss


# speedup vs baseline: 1.4902x; 1.4902x over previous
import functools

import jax
import jax.numpy as jnp
from jax import lax
from jax.experimental import pallas as pl
from jax.experimental.pallas import tpu as pltpu
from jax.experimental.pallas import tpu_sc as plsc

_NC = 2
_NS = 16
_NW = _NC * _NS
_CHUNK = 128


def _vq_tc_body(h_ref, w_ref, idx_ref, loss_ref):
    i = pl.program_id(0)
    hb = h_ref[...]
    w = w_ref[...]
    K = w.shape[0]
    D = hb.shape[1]
    w_sq = jnp.sum(w * w, axis=1, keepdims=True)
    m2 = jax.lax.dot_general(-2.0 * w, hb, (((1,), (1,)), ((), ())),
                             preferred_element_type=jnp.float32)
    score = w_sq + m2
    minval = jnp.min(score, axis=0, keepdims=True)
    iota_f = jax.lax.broadcasted_iota(
        jnp.int32, score.shape, 0).astype(jnp.float32)
    idx_f = jnp.min(jnp.where(score == minval, iota_f, float(K)), axis=0)
    BN = idx_f.shape[0]
    idx_ref[0, :, :] = idx_f.astype(jnp.int32).reshape(BN // 128, 128)

    h_sq = jax.lax.dot_general(
        jnp.ones((1, D), jnp.float32), hb * hb,
        (((1,), (1,)), ((), ())),
        preferred_element_type=jnp.float32)

    @pl.when(i == 0)
    def _():
        loss_ref[0, 0] = 0.0

    loss_ref[0, 0] += jnp.sum(h_sq + minval)


def _tc_stage(h_flat, W, BN, off, rows):
    N, D = h_flat.shape
    K = W.shape[0]
    grid = rows // BN
    off_blocks = off // BN

    idx3, loss_sum = pl.pallas_call(
        _vq_tc_body,
        grid=(grid,),
        in_specs=[
            pl.BlockSpec((BN, D), lambda i: (i + off_blocks, 0)),
            pl.BlockSpec((K, D), lambda i: (0, 0)),
        ],
        out_specs=[
            pl.BlockSpec((1, BN // 128, 128), lambda i: (i, 0, 0)),
            pl.BlockSpec((1, 1), lambda i: (0, 0), memory_space=pltpu.SMEM),
        ],
        out_shape=[
            jax.ShapeDtypeStruct((grid, BN // 128, 128), jnp.int32),
            jax.ShapeDtypeStruct((1, 1), jnp.float32),
        ],
        compiler_params=pltpu.CompilerParams(
            dimension_semantics=("arbitrary",)),
    )(h_flat, W)
    return idx3.reshape(rows), loss_sum


def _make_sc_gather(N, K, D):
    b_per_w = N // _NW
    n_chunks = b_per_w // _CHUNK
    mesh = plsc.VectorSubcoreMesh(core_axis_name="c", subcore_axis_name="s")

    @functools.partial(
        pl.kernel,
        mesh=mesh,
        out_type=jax.ShapeDtypeStruct((N, D), jnp.float32),
        compiler_params=pltpu.CompilerParams(use_tc_tiling_on_sc=False),
        scratch_types=[
            pltpu.VMEM((b_per_w,), jnp.int32),
            pltpu.VMEM((b_per_w, D), jnp.float32),
            pltpu.SemaphoreType.DMA,
            pltpu.SemaphoreType.DMA,
        ],
    )
    def gather_kernel(idx_hbm, table_hbm, out_hbm, idx_v, rows_v, gsem,
                      wsem):
        wid = lax.axis_index("s") * _NC + lax.axis_index("c")
        base = wid * b_per_w
        pltpu.sync_copy(idx_hbm.at[pl.ds(base, b_per_w)], idx_v)
        copies = []
        for c in range(n_chunks):
            copies.append(pltpu.async_copy(
                table_hbm.at[idx_v.at[pl.ds(c * _CHUNK, _CHUNK)]],
                rows_v.at[pl.ds(c * _CHUNK, _CHUNK)],
                gsem))
        wb = []
        for c in range(n_chunks):
            copies[c].wait()
            wb.append(pltpu.async_copy(
                rows_v.at[pl.ds(c * _CHUNK, _CHUNK)],
                out_hbm.at[pl.ds(base + c * _CHUNK, _CHUNK)],
                wsem))
        for cp in wb:
            cp.wait()

    return gather_kernel


def kernel(h, W):
    N = h.shape[0] * h.shape[1]
    D = h.shape[2]
    K = W.shape[0]
    h_flat = h.reshape(N, D)

    idx, loss_sum = _tc_stage(h_flat, W, 2048, 0, N)
    q = _make_sc_gather(N, K, D)(idx, W)

    mse = loss_sum[0, 0] / jnp.float32(N * D)
    commitment_loss = jnp.float32(0.25) * mse
    codebook_loss = mse
    return q.reshape(h.shape), commitment_loss, codebook_loss


# speedup vs baseline: 1.5632x; 1.0490x over previous
import functools

import jax
import jax.numpy as jnp
from jax import lax
from jax.experimental import pallas as pl
from jax.experimental.pallas import tpu as pltpu
from jax.experimental.pallas import tpu_sc as plsc

_NC = 2
_NS = 16
_NW = _NC * _NS
_CHUNK = 128


def _vq_tc_body(h_ref, w_ref, idx_ref, loss_ref):
    i = pl.program_id(0)
    hb = h_ref[...]
    w = w_ref[...]
    K = w.shape[0]
    D = hb.shape[1]
    w_sq = jnp.sum(w * w, axis=1, keepdims=True)
    m2 = jax.lax.dot_general(-2.0 * w, hb, (((1,), (1,)), ((), ())),
                             preferred_element_type=jnp.float32)
    score = w_sq + m2
    minval = jnp.min(score, axis=0, keepdims=True)
    iota_f = jax.lax.broadcasted_iota(
        jnp.int32, score.shape, 0).astype(jnp.float32)
    idx_f = jnp.min(jnp.where(score == minval, iota_f, float(K)), axis=0)
    BN = idx_f.shape[0]
    idx_ref[0, :, :] = idx_f.astype(jnp.int32).reshape(BN // 128, 128)

    h_sq = jax.lax.dot_general(
        jnp.ones((1, D), jnp.float32), hb * hb,
        (((1,), (1,)), ((), ())),
        preferred_element_type=jnp.float32)

    @pl.when(i == 0)
    def _():
        loss_ref[0, 0] = 0.0

    loss_ref[0, 0] += jnp.sum(h_sq + minval)


def _tc_stage(h_flat, W, BN, off, rows):
    N, D = h_flat.shape
    K = W.shape[0]
    grid = rows // BN
    off_blocks = off // BN

    idx3, loss_sum = pl.pallas_call(
        _vq_tc_body,
        grid=(grid,),
        in_specs=[
            pl.BlockSpec((BN, D), lambda i: (i + off_blocks, 0)),
            pl.BlockSpec((K, D), lambda i: (0, 0)),
        ],
        out_specs=[
            pl.BlockSpec((1, BN // 128, 128), lambda i: (i, 0, 0)),
            pl.BlockSpec((1, 1), lambda i: (0, 0), memory_space=pltpu.SMEM),
        ],
        out_shape=[
            jax.ShapeDtypeStruct((grid, BN // 128, 128), jnp.int32),
            jax.ShapeDtypeStruct((1, 1), jnp.float32),
        ],
        compiler_params=pltpu.CompilerParams(
            dimension_semantics=("arbitrary",)),
    )(h_flat, W)
    return idx3.reshape(rows), loss_sum


def _make_sc_gather(N, K, D):
    b_per_w = N // _NW
    n_chunks = b_per_w // _CHUNK
    mesh = plsc.VectorSubcoreMesh(core_axis_name="c", subcore_axis_name="s")

    @functools.partial(
        pl.kernel,
        mesh=mesh,
        out_type=jax.ShapeDtypeStruct((N, D), jnp.float32),
        compiler_params=pltpu.CompilerParams(use_tc_tiling_on_sc=False),
        scratch_types=[
            pltpu.VMEM((b_per_w,), jnp.int32),
            pltpu.VMEM((b_per_w, D), jnp.float32),
            pltpu.SemaphoreType.DMA,
            pltpu.SemaphoreType.DMA,
        ],
    )
    def gather_kernel(idx_hbm, table_hbm, out_hbm, idx_v, rows_v, gsem,
                      wsem):
        wid = lax.axis_index("s") * _NC + lax.axis_index("c")
        base = wid * b_per_w
        pltpu.sync_copy(idx_hbm.at[pl.ds(base, b_per_w)], idx_v)
        copies = []
        for c in range(n_chunks):
            copies.append(pltpu.async_copy(
                table_hbm.at[idx_v.at[pl.ds(c * _CHUNK, _CHUNK)]],
                rows_v.at[pl.ds(c * _CHUNK, _CHUNK)],
                gsem))
        for cp in copies:
            cp.wait()
        pltpu.sync_copy(rows_v, out_hbm.at[pl.ds(base, b_per_w)])

    return gather_kernel


def kernel(h, W):
    N = h.shape[0] * h.shape[1]
    D = h.shape[2]
    K = W.shape[0]
    h_flat = h.reshape(N, D)

    idx, loss_sum = _tc_stage(h_flat, W, 4096, 0, N)
    q = _make_sc_gather(N, K, D)(idx, W)

    mse = loss_sum[0, 0] / jnp.float32(N * D)
    commitment_loss = jnp.float32(0.25) * mse
    codebook_loss = mse
    return q.reshape(h.shape), commitment_loss, codebook_loss
